# baseline (speedup 1.0000x reference)
import jax
import jax.numpy as jnp
from jax.experimental import pallas as pl

_OUT = 65
_L = 2048
_H = 128
_MROWS = 72
_R = 32


def _fused_kernel(x_ref, m_ref, mask_ref, out_ref):
    xb = x_ref[...]
    nz = xb != 0.0
    idx = jax.lax.broadcasted_iota(jnp.int32, xb.shape, 1)
    first = jnp.min(jnp.where(nz, idx, _L), axis=1, keepdims=True)
    last = jnp.max(jnp.where(nz, idx, -1), axis=1, keepdims=True)
    mask_ref[...] = (idx >= first) & (idx <= last)
    out_ref[...] = m_ref[:, :_OUT, :_OUT]


def kernel(x, matrix):
    N, C, L = x.shape
    rows = N * C
    x2 = x.reshape(rows, L)
    m2 = matrix.reshape(rows, _H, _H)
    mask, resized = pl.pallas_call(
        _fused_kernel,
        grid=(rows // _R,),
        in_specs=[
            pl.BlockSpec((_R, L), lambda i: (i, 0)),
            pl.BlockSpec((_R, _MROWS, _H), lambda i: (i, 0, 0)),
        ],
        out_specs=[
            pl.BlockSpec((_R, L), lambda i: (i, 0)),
            pl.BlockSpec((_R, _OUT, _OUT), lambda i: (i, 0, 0)),
        ],
        out_shape=[
            jax.ShapeDtypeStruct((rows, L), jnp.bool_),
            jax.ShapeDtypeStruct((rows, _OUT, _OUT), jnp.float32),
        ],
    )(x2, m2)
    return mask.reshape(N, C, L), resized.reshape(N, C, _OUT, _OUT)
